# baseline (speedup 1.0000x reference)
import functools

import jax
import jax.numpy as jnp
from jax import lax
from jax.experimental import pallas as pl
from jax.experimental.pallas import tpu as pltpu
from jax.experimental.pallas import tpu_sc as plsc

_NC, _NS = 2, 16
_NW = _NC * _NS
_NSC = 5120
_RC = 8
_B1 = 488
_B2 = 80


def _sc_rowsum(nbr, row0, n_sc):
    n, k, d = nbr.shape
    rpw = n_sc // _NW
    nchunks = rpw // _RC
    mesh = plsc.VectorSubcoreMesh(core_axis_name="c", subcore_axis_name="s")

    @functools.partial(
        pl.kernel, mesh=mesh,
        out_type=jax.ShapeDtypeStruct((n_sc, d), jnp.float32),
        scratch_types=[
            pltpu.VMEM((2, _RC, k, d), jnp.float32),
            pltpu.VMEM((2, _RC, d), jnp.float32),
            pltpu.SemaphoreType.DMA,
            pltpu.SemaphoreType.DMA,
        ],
    )
    def body(nbr_hbm, out_hbm, inbuf, outbuf, sem0, sem1):
        wid = lax.axis_index("s") * _NC + lax.axis_index("c")
        base = row0 + wid * rpw
        obase = wid * rpw
        sems = (sem0, sem1)

        pltpu.async_copy(nbr_hbm.at[pl.ds(base, _RC)], inbuf.at[0], sem0)
        pltpu.async_copy(nbr_hbm.at[pl.ds(base + _RC, _RC)], inbuf.at[1], sem1)

        def half_step(c2, carry):
            for b in range(2):
                chunk = 2 * c2 + b
                pltpu.make_async_copy(
                    nbr_hbm.at[pl.ds(0, _RC)], inbuf.at[b], sems[b]).wait()
                for r in range(_RC):
                    for j in range(d // 16):
                        acc = inbuf[b, r, 0, pl.ds(j * 16, 16)]
                        for kk in range(1, k):
                            acc = acc + inbuf[b, r, kk, pl.ds(j * 16, 16)]
                        outbuf[b, r, pl.ds(j * 16, 16)] = acc
                @pl.when(chunk + 2 < nchunks)
                def _():
                    pltpu.async_copy(
                        nbr_hbm.at[pl.ds(base + (chunk + 2) * _RC, _RC)],
                        inbuf.at[b], sems[b])
                pltpu.sync_copy(outbuf.at[b],
                                out_hbm.at[pl.ds(obase + chunk * _RC, _RC)])
            return carry

        lax.fori_loop(0, nchunks // 2, half_step, 0)

    return body(nbr)


def _tc_fused_body(src_ref, nbr_ref, idx_ref, wagg_ref, bagg_ref, wself_ref, out_ref):
    seq = jnp.sum((idx_ref[...] != -1).astype(jnp.float32), axis=1)
    aggr = jnp.sum(nbr_ref[...], axis=1) / seq[:, None]
    nh = jnp.dot(aggr, wagg_ref[...], preferred_element_type=jnp.float32)
    sh = jnp.dot(src_ref[...], wself_ref[...], preferred_element_type=jnp.float32)
    h = sh + nh + bagg_ref[...]
    out_ref[...] = jnp.where(h >= 0, h, 0.01 * h)


def _tc_finish_body(src_ref, sum_ref, idx_ref, wagg_ref, bagg_ref, wself_ref, out_ref):
    seq = jnp.sum((idx_ref[...] != -1).astype(jnp.float32), axis=1)
    aggr = sum_ref[...] / seq[:, None]
    nh = jnp.dot(aggr, wagg_ref[...], preferred_element_type=jnp.float32)
    sh = jnp.dot(src_ref[...], wself_ref[...], preferred_element_type=jnp.float32)
    h = sh + nh + bagg_ref[...]
    out_ref[...] = jnp.where(h >= 0, h, 0.01 * h)


@jax.jit
def kernel(src_node_features, neighbor_node_features, neighbor_node_idx, W_agg, b_agg, W_self):
    n, k, d = neighbor_node_features.shape
    h = W_agg.shape[1]
    nt = n - _NSC
    bagg2d = b_agg.reshape(1, h)

    sc_sums = _sc_rowsum(neighbor_node_features, nt, _NSC)

    out1 = pl.pallas_call(
        _tc_fused_body,
        grid=(nt // _B1,),
        in_specs=[
            pl.BlockSpec((_B1, d), lambda i: (i, 0)),
            pl.BlockSpec((_B1, k, d), lambda i: (i, 0, 0)),
            pl.BlockSpec((_B1, k), lambda i: (i, 0)),
            pl.BlockSpec((d, h), lambda i: (0, 0)),
            pl.BlockSpec((1, h), lambda i: (0, 0)),
            pl.BlockSpec((d, h), lambda i: (0, 0)),
        ],
        out_specs=pl.BlockSpec((_B1, h), lambda i: (i, 0)),
        out_shape=jax.ShapeDtypeStruct((nt, h), jnp.float32),
        compiler_params=pltpu.CompilerParams(
            dimension_semantics=("parallel",),
        ),
    )(src_node_features, neighbor_node_features, neighbor_node_idx,
      W_agg, bagg2d, W_self)

    nt_b2 = nt // _B2
    out2 = pl.pallas_call(
        _tc_finish_body,
        grid=(_NSC // _B2,),
        in_specs=[
            pl.BlockSpec((_B2, d), lambda i: (i + nt_b2, 0)),
            pl.BlockSpec((_B2, d), lambda i: (i, 0)),
            pl.BlockSpec((_B2, k), lambda i: (i + nt_b2, 0)),
            pl.BlockSpec((d, h), lambda i: (0, 0)),
            pl.BlockSpec((1, h), lambda i: (0, 0)),
            pl.BlockSpec((d, h), lambda i: (0, 0)),
        ],
        out_specs=pl.BlockSpec((_B2, h), lambda i: (i, 0)),
        out_shape=jax.ShapeDtypeStruct((_NSC, h), jnp.float32),
        compiler_params=pltpu.CompilerParams(
            dimension_semantics=("parallel",),
        ),
    )(src_node_features, sc_sums, neighbor_node_idx,
      W_agg, bagg2d, W_self)

    return jnp.concatenate([out1, out2], axis=0)
